# Initial kernel scaffold; baseline (speedup 1.0000x reference)
import jax
import jax.numpy as jnp
from jax.experimental import pallas as pl


def kernel(x, edge_index, Wl1, bl1, Wr1, Wl2, bl2, Wr2, Wlin, blin):
    raise NotImplementedError("write your pallas kernel here")


# speedup vs baseline: 3.0548x; 3.0548x over previous
import jax
import jax.numpy as jnp
from jax import lax
from jax.experimental import pallas as pl
from jax.experimental.pallas import tpu as pltpu
from jax.experimental.pallas import tpu_sc as plsc

N = 10000
E = 160000
D = 256
DH = 128
K = 128
NTILES = 16
NCH = 79
EPT = NCH * K
E_PAD = NTILES * EPT
N_ACC = 10240
ROWS_PER_TILE = N_ACC // NTILES


def _make_sc_agg(with_deg):
    def body(*refs):
        if with_deg:
            (table, srcp, dstp, zrows, out_sum, out_deg,
             acc_sp, stage_sp, src_v, dst_v, dst1_v, idx2_v, rows_v,
             deg_loc, tmp_v, sem) = refs
        else:
            (table, srcp, dstp, zrows, out_sum,
             acc_sp, src_v, dst_v, idx2_v, rows_v, sem) = refs

        c = lax.axis_index("c")
        s = lax.axis_index("s")

        r0 = pl.multiple_of(s * ROWS_PER_TILE, ROWS_PER_TILE)
        co = pl.multiple_of(c * DH, DH)
        pltpu.sync_copy(zrows, rows_v)
        for k in range(ROWS_PER_TILE // K):
            pltpu.sync_copy(rows_v, acc_sp.at[pl.ds(r0 + k * K, K), :])
        if with_deg:
            zero16 = jnp.zeros((16,), jnp.float32)

            def zbody(i, carry):
                deg_loc[pl.ds(i * 16, 16)] = zero16
                return carry

            lax.fori_loop(0, N_ACC // 16, zbody, 0)

        plsc.subcore_barrier()

        base = s * EPT
        ones16 = jnp.ones((16,), jnp.float32)

        def chunk(ch, carry):
            off = base + ch * K
            pltpu.sync_copy(srcp.at[pl.ds(off, K)], src_v)
            pltpu.sync_copy(dstp.at[pl.ds(off, K)], dst_v.at[0])
            for i in range(K // 16):
                sl = pl.ds(i * 16, 16)
                idx2_v[sl] = src_v[sl] * 2 + c
            pltpu.async_copy(table.at[idx2_v], rows_v, sem).wait()
            pltpu.sync_copy(rows_v, acc_sp.at[dst_v.at[0]], add=True)
            if with_deg:
                pltpu.sync_copy(dstp.at[pl.ds(off, K)], dst1_v)
                for i in range(K // 16):
                    sl = pl.ds(i * 16, 16)
                    plsc.addupdate_scatter(deg_loc, [dst1_v[sl]], ones16)
            return carry

        lax.fori_loop(0, NCH, chunk, 0)

        if with_deg:
            def addbody(i, carry):
                sl = pl.ds(i * 16, 16)
                deg_loc[sl] = deg_loc[sl] + tmp_v[sl]
                return carry

            for r in (1, 2, 4, 8):
                pltpu.sync_copy(deg_loc, stage_sp.at[s])
                plsc.subcore_barrier()
                pltpu.sync_copy(stage_sp.at[lax.bitwise_xor(s, r)], tmp_v)
                plsc.subcore_barrier()
                lax.fori_loop(0, N_ACC // 16, addbody, 0)
            o0 = pl.multiple_of(s * ROWS_PER_TILE + c * (ROWS_PER_TILE // 2),
                                ROWS_PER_TILE // 2)
            pltpu.sync_copy(deg_loc.at[pl.ds(o0, ROWS_PER_TILE // 2)],
                            out_deg.at[pl.ds(o0, ROWS_PER_TILE // 2)])

        plsc.subcore_barrier()

        for k in range(ROWS_PER_TILE // K):
            rr = r0 + k * K
            pltpu.sync_copy(acc_sp.at[pl.ds(rr, K), :], rows_v)
            pltpu.sync_copy(rows_v, out_sum.at[pl.ds(rr, K), pl.ds(co, DH)])

    if with_deg:
        out_type = (jax.ShapeDtypeStruct((N_ACC, D), jnp.float32),
                    jax.ShapeDtypeStruct((N_ACC,), jnp.float32))
    else:
        out_type = jax.ShapeDtypeStruct((N_ACC, D), jnp.float32)
    scratch = [pltpu.VMEM_SHARED((N_ACC, DH), jnp.float32)]
    if with_deg:
        scratch.append(pltpu.VMEM_SHARED((NTILES, N_ACC), jnp.float32))
    scratch += [
        pltpu.VMEM((K,), jnp.int32),
        pltpu.VMEM((1, K), jnp.int32),
    ]
    if with_deg:
        scratch.append(pltpu.VMEM((K,), jnp.int32))
    scratch += [
        pltpu.VMEM((K,), jnp.int32),
        pltpu.VMEM((K, DH), jnp.float32),
    ]
    if with_deg:
        scratch.append(pltpu.VMEM((N_ACC,), jnp.float32))
        scratch.append(pltpu.VMEM((N_ACC,), jnp.float32))
    scratch.append(pltpu.SemaphoreType.DMA)
    return pl.kernel(
        body,
        out_type=out_type,
        mesh=plsc.VectorSubcoreMesh(core_axis_name="c", subcore_axis_name="s"),
        scratch_types=scratch,
        compiler_params=pltpu.CompilerParams(needs_layout_passes=False),
    )


_sc_agg_deg = _make_sc_agg(with_deg=True)
_sc_agg = _make_sc_agg(with_deg=False)


BLK = 1000


def _tc1_body(deg_ref, s_ref, x_ref, wl_ref, wr_ref, bl_ref, o_ref):
    r = 1.0 / jnp.maximum(deg_ref[...], 1.0)
    mean = s_ref[...] * r
    acc = lax.dot_general(mean, wl_ref[...], (((1,), (1,)), ((), ())),
                          preferred_element_type=jnp.float32)
    acc = acc + lax.dot_general(x_ref[...], wr_ref[...], (((1,), (1,)), ((), ())),
                                preferred_element_type=jnp.float32)
    o_ref[...] = jnp.maximum(acc + bl_ref[...], 0.0)


def _tc2_body(deg_ref, s_ref, x1_ref, wl_ref, wr_ref, bl_ref, wlin_ref,
              blin_ref, o_ref):
    r = 1.0 / jnp.maximum(deg_ref[...], 1.0)
    mean = s_ref[...] * r
    acc = lax.dot_general(mean, wl_ref[...], (((1,), (1,)), ((), ())),
                          preferred_element_type=jnp.float32)
    acc = acc + lax.dot_general(x1_ref[...], wr_ref[...], (((1,), (1,)), ((), ())),
                                preferred_element_type=jnp.float32)
    x2 = jnp.maximum(acc + bl_ref[...], 0.0)
    z = lax.dot_general(x1_ref[...], wlin_ref[:, :D], (((1,), (1,)), ((), ())),
                        preferred_element_type=jnp.float32)
    z = z + lax.dot_general(x2, wlin_ref[:, D:], (((1,), (1,)), ((), ())),
                            preferred_element_type=jnp.float32)
    o_ref[...] = jax.nn.sigmoid(z + blin_ref[...])


def _tc_layer1(deg16, s, x, Wl, Wr, bl):
    return pl.pallas_call(
        _tc1_body,
        grid=(N // BLK,),
        in_specs=[
            pl.BlockSpec((BLK, 1), lambda i: (i, 0)),
            pl.BlockSpec((BLK, D), lambda i: (i, 0)),
            pl.BlockSpec((BLK, D), lambda i: (i, 0)),
            pl.BlockSpec((D, D), lambda i: (0, 0)),
            pl.BlockSpec((D, D), lambda i: (0, 0)),
            pl.BlockSpec((1, D), lambda i: (0, 0)),
        ],
        out_specs=pl.BlockSpec((BLK, D), lambda i: (i, 0)),
        out_shape=jax.ShapeDtypeStruct((N, D), jnp.float32),
    )(deg16, s, x, Wl, Wr, bl)


def _tc_layer2(deg16, s, x1, Wl, Wr, bl, Wlin, blin):
    return pl.pallas_call(
        _tc2_body,
        grid=(N // BLK,),
        in_specs=[
            pl.BlockSpec((BLK, 1), lambda i: (i, 0)),
            pl.BlockSpec((BLK, D), lambda i: (i, 0)),
            pl.BlockSpec((BLK, D), lambda i: (i, 0)),
            pl.BlockSpec((D, D), lambda i: (0, 0)),
            pl.BlockSpec((D, D), lambda i: (0, 0)),
            pl.BlockSpec((1, D), lambda i: (0, 0)),
            pl.BlockSpec((1, 2 * D), lambda i: (0, 0)),
            pl.BlockSpec((1, 1), lambda i: (0, 0)),
        ],
        out_specs=pl.BlockSpec((BLK, 1), lambda i: (i, 0)),
        out_shape=jax.ShapeDtypeStruct((N, 1), jnp.float32),
    )(deg16, s, x1, Wl, Wr, bl, Wlin, blin)


def kernel(x, edge_index, Wl1, bl1, Wr1, Wl2, bl2, Wr2, Wlin, blin):
    src = edge_index[0].astype(jnp.int32)
    dst = edge_index[1].astype(jnp.int32)
    pad = E_PAD - E
    srcp = jnp.concatenate([src, jnp.zeros((pad,), jnp.int32)])
    dstp = jnp.concatenate([dst, jnp.full((pad,), N, jnp.int32)])
    zrows = jnp.zeros((K, DH), jnp.float32)

    bl1r = bl1.reshape(1, D)
    bl2r = bl2.reshape(1, D)
    blinr = blin.reshape(1, 1)

    sum1, deg = _sc_agg_deg(x.reshape(2 * N, DH), srcp, dstp, zrows)
    degc = deg.reshape(N_ACC, 1)
    x1 = _tc_layer1(degc, sum1, x, Wl1, Wr1, bl1r)
    sum2 = _sc_agg(x1.reshape(2 * N, DH), srcp, dstp, zrows)
    return _tc_layer2(degc, sum2, x1, Wl2, Wr2, bl2r, Wlin, blinr)
